# baseline (speedup 1.0000x reference)
import functools

import jax
import jax.numpy as jnp
from jax import lax
from jax.experimental import pallas as pl
from jax.experimental.pallas import tpu as pltpu
from jax.experimental.pallas import tpu_sc as plsc

_VOCAB = 1000000
_D = 64
_B = 4096
_L = 200
_NC = 2
_NS = 16
_NW = _NC * _NS
_LANES = 16
_SCALE = 8.0

_VBLK = 128
_NVB = -(-_VOCAB // _VBLK)
_A_ITERS = -(-_NVB // _NW)
_A_NBUF = 4
_VLAST = _VOCAB - (_NVB - 1) * _VBLK

_BBLK = _B // _NW
_B_NBUF = 4
_B_OBUF = 2
_B_NGROUP = _L // _B_NBUF
_SLAB = 40


def _prep_body(tt_hbm, tp_hbm, *rest):
    ibufs = rest[0:_A_NBUF]
    obufs = rest[_A_NBUF:2 * _A_NBUF]
    isems = rest[2 * _A_NBUF:3 * _A_NBUF]
    osems = rest[3 * _A_NBUF:4 * _A_NBUF]

    wid = lax.axis_index("s") * _NC + lax.axis_index("c")
    iota = lax.iota(jnp.int32, _LANES)

    def blk(it):
        return wid + it * _NW

    def in_copy(it, b):
        return pltpu.make_async_copy(
            tt_hbm.at[:, pl.ds(blk(it) * _VBLK, _VBLK)],
            ibufs[b].at[:, pl.ds(0, _VBLK)], isems[b])

    def out_copy(it, b):
        return pltpu.make_async_copy(
            obufs[b], tp_hbm.at[pl.ds(blk(it) * _VBLK, _VBLK)], osems[b])

    def out_copy_last(it, b):
        return pltpu.make_async_copy(
            obufs[b].at[pl.ds(0, _VLAST)],
            tp_hbm.at[pl.ds(blk(it) * _VBLK, _VLAST)], osems[b])

    for b in range(_A_NBUF):
        @pl.when(blk(b) < _NVB)
        def _(b=b):
            in_copy(b, b).start()

    def step(it, b):
        bi = blk(it)

        @pl.when(bi < _NVB)
        def _():
            in_copy(it, b).wait()

            @pl.when(it >= _A_NBUF)
            def _():
                @pl.when(blk(it - _A_NBUF) < _NVB - 1)
                def _():
                    out_copy(it - _A_NBUF, b).wait()

                @pl.when(blk(it - _A_NBUF) == _NVB - 1)
                def _():
                    out_copy_last(it - _A_NBUF, b).wait()

            ibuf = ibufs[b]
            obuf = obufs[b]

            zero = iota * 0

            @plsc.parallel_loop(0, _VBLK, unroll=4)
            def _row(v, ibuf=ibuf, obuf=obuf, zero=zero):
                vi = zero + v
                for kk in range(_D // _LANES):
                    di = iota + (kk * _LANES)
                    vals = plsc.load_gather(ibuf, [di, vi])
                    obuf[v, pl.ds(kk * _LANES, _LANES)] = vals * _SCALE

            @pl.when(bi < _NVB - 1)
            def _():
                out_copy(it, b).start()

            @pl.when(bi == _NVB - 1)
            def _():
                out_copy_last(it, b).start()

            @pl.when(blk(it + _A_NBUF) < _NVB)
            def _():
                in_copy(it + _A_NBUF, b).start()

    def group(g, carry):
        for b in range(_A_NBUF):
            step(g * _A_NBUF + b, b)
        return carry

    lax.fori_loop(0, -(-_A_ITERS // _A_NBUF), group, 0)

    last_groups = -(-_A_ITERS // _A_NBUF)
    for b in range(_A_NBUF):
        it = (last_groups - 1) * _A_NBUF + b
        for back in range(2):
            itb = it - back * _A_NBUF
            bi_tb = blk(itb)
            pending = (bi_tb < _NVB) & (bi_tb + _NW * _A_NBUF >= _NVB)

            @pl.when(pending & (bi_tb < _NVB - 1))
            def _(itb=itb, b=b):
                out_copy(itb, b).wait()

            @pl.when(pending & (bi_tb == _NVB - 1))
            def _(itb=itb, b=b):
                out_copy_last(itb, b).wait()


_prep_call = functools.partial(
    pl.kernel,
    out_type=jax.ShapeDtypeStruct((_VOCAB, 2 * _D), jnp.float32),
    mesh=plsc.VectorSubcoreMesh(core_axis_name="c", subcore_axis_name="s"),
    compiler_params=pltpu.CompilerParams(needs_layout_passes=False),
    scratch_types=(
        [pltpu.VMEM((_D, _VBLK + 1), jnp.float32) for _ in range(_A_NBUF)]
        + [pltpu.VMEM((_VBLK, 2 * _D), jnp.float32) for _ in range(_A_NBUF)]
        + [pltpu.SemaphoreType.DMA for _ in range(2 * _A_NBUF)]
    ),
)(_prep_body)


def _lookup_body(tp_hbm, xt_hbm, out_hbm, idx_v, *rest):
    gbufs = rest[0:_B_NBUF]
    obufs = rest[_B_NBUF:_B_NBUF + _B_OBUF]
    gsems = rest[_B_NBUF + _B_OBUF:2 * _B_NBUF + _B_OBUF]
    osems = rest[2 * _B_NBUF + _B_OBUF:2 * _B_NBUF + 2 * _B_OBUF]

    wid = lax.axis_index("s") * _NC + lax.axis_index("c")
    col0 = wid * _BBLK
    iota = lax.iota(jnp.int32, _LANES)

    pltpu.sync_copy(xt_hbm.at[pl.ds(0, _SLAB), pl.ds(col0, _BBLK)], idx_v)

    def gather_copy(c, b):
        return pltpu.make_async_copy(
            tp_hbm.at[idx_v.at[c % _SLAB]], gbufs[b], gsems[b])

    def out_start(c, ob):
        for dt in range(_D // 8):
            pltpu.make_async_copy(
                obufs[ob].at[pl.ds(8 * dt, 8), pl.ds(0, _BBLK)],
                out_hbm.at[c, pl.ds(8 * dt, 8), pl.ds(col0, _BBLK)],
                osems[ob]).start()

    def out_wait(c, ob):
        for dt in range(_D // 8):
            pltpu.make_async_copy(
                obufs[ob].at[pl.ds(8 * dt, 8), pl.ds(0, _BBLK)],
                out_hbm.at[c, pl.ds(8 * dt, 8), pl.ds(col0, _BBLK)],
                osems[ob]).wait()

    for b in range(_B_NBUF):
        gather_copy(b, b).start()

    def group(g, carry):
        @pl.when(((g + 1) % (_SLAB // _B_NBUF) == 0)
                 & (g + 1 < _B_NGROUP))
        def _():
            sidx = (g + 1) // (_SLAB // _B_NBUF)
            pltpu.sync_copy(
                xt_hbm.at[pl.ds(sidx * _SLAB, _SLAB), pl.ds(col0, _BBLK)],
                idx_v)

        for b in range(_B_NBUF):
            c = g * _B_NBUF + b
            ob = b % _B_OBUF
            gather_copy(c, b).wait()

            @pl.when(c >= _B_OBUF)
            def _():
                out_wait(c - _B_OBUF, ob)

            gbuf = gbufs[b]
            obuf = obufs[ob]

            @plsc.parallel_loop(0, _BBLK, unroll=4)
            def _row(j, gbuf=gbuf, obuf=obuf):
                ji = iota * 0 + j
                for kk in range(_D // _LANES):
                    vals = gbuf[j, pl.ds(kk * _LANES, _LANES)]
                    plsc.store_scatter(obuf, [iota + kk * _LANES, ji], vals)

            out_start(c, ob)

            @pl.when(g + 1 < _B_NGROUP)
            def _():
                gather_copy(c + _B_NBUF, b).start()
        return carry

    lax.fori_loop(0, _B_NGROUP, group, 0)

    for b in range(_B_OBUF):
        c = _L - _B_OBUF + b
        out_wait(c, c % _B_OBUF)


_lookup_call = functools.partial(
    pl.kernel,
    out_type=jax.ShapeDtypeStruct((_L, _D, _B), jnp.float32),
    mesh=plsc.VectorSubcoreMesh(core_axis_name="c", subcore_axis_name="s"),
    compiler_params=pltpu.CompilerParams(needs_layout_passes=False),
    scratch_types=(
        [pltpu.VMEM((_SLAB, _BBLK), jnp.int32)]
        + [pltpu.VMEM((_BBLK, 2 * _D), jnp.float32) for _ in range(_B_NBUF)]
        + [pltpu.VMEM((_D, _BBLK + 1), jnp.float32) for _ in range(_B_OBUF)]
        + [pltpu.SemaphoreType.DMA for _ in range(_B_NBUF + _B_OBUF)]
    ),
)(_lookup_body)


@jax.jit
def kernel(embedding_table, x):
    tp = _prep_call(embedding_table.T)
    out = _lookup_call(tp, x.T)
    return out.transpose(2, 0, 1)


# speedup vs baseline: 1.0015x; 1.0015x over previous
import functools

import jax
import jax.numpy as jnp
from jax import lax
from jax.experimental import pallas as pl
from jax.experimental.pallas import tpu as pltpu
from jax.experimental.pallas import tpu_sc as plsc

_VOCAB = 1000000
_D = 64
_B = 4096
_L = 200
_NC = 2
_NS = 16
_NW = _NC * _NS
_LANES = 16
_SCALE = 8.0

_VBLK = 128
_NVB = -(-_VOCAB // _VBLK)
_A_ITERS = -(-_NVB // _NW)
_A_NBUF = 4
_VLAST = _VOCAB - (_NVB - 1) * _VBLK

_BBLK = _B // _NW
_B_NBUF = 4
_B_OBUF = 2
_B_NGROUP = _L // _B_NBUF
_SLAB = 40


def _prep_body(tt_hbm, tp_hbm, *rest):
    ibufs = rest[0:_A_NBUF]
    obufs = rest[_A_NBUF:2 * _A_NBUF]
    isems = rest[2 * _A_NBUF:3 * _A_NBUF]
    osems = rest[3 * _A_NBUF:4 * _A_NBUF]

    wid = lax.axis_index("s") * _NC + lax.axis_index("c")
    iota = lax.iota(jnp.int32, _LANES)

    def blk(it):
        return wid + it * _NW

    def in_copy(it, b):
        return pltpu.make_async_copy(
            tt_hbm.at[:, pl.ds(blk(it) * _VBLK, _VBLK)],
            ibufs[b].at[:, pl.ds(0, _VBLK)], isems[b])

    def out_copy(it, b):
        return pltpu.make_async_copy(
            obufs[b], tp_hbm.at[pl.ds(blk(it) * _VBLK, _VBLK)], osems[b])

    def out_copy_last(it, b):
        return pltpu.make_async_copy(
            obufs[b].at[pl.ds(0, _VLAST)],
            tp_hbm.at[pl.ds(blk(it) * _VBLK, _VLAST)], osems[b])

    for b in range(_A_NBUF):
        @pl.when(blk(b) < _NVB)
        def _(b=b):
            in_copy(b, b).start()

    def step(it, b):
        bi = blk(it)

        @pl.when(bi < _NVB)
        def _():
            in_copy(it, b).wait()

            @pl.when(it >= _A_NBUF)
            def _():
                @pl.when(blk(it - _A_NBUF) < _NVB - 1)
                def _():
                    out_copy(it - _A_NBUF, b).wait()

                @pl.when(blk(it - _A_NBUF) == _NVB - 1)
                def _():
                    out_copy_last(it - _A_NBUF, b).wait()

            ibuf = ibufs[b]
            obuf = obufs[b]

            zero = iota * 0

            @plsc.parallel_loop(0, _VBLK, unroll=8)
            def _row(v, ibuf=ibuf, obuf=obuf, zero=zero):
                vi = zero + v
                for kk in range(_D // _LANES):
                    di = iota + (kk * _LANES)
                    vals = plsc.load_gather(ibuf, [di, vi])
                    obuf[v, pl.ds(kk * _LANES, _LANES)] = vals * _SCALE

            @pl.when(bi < _NVB - 1)
            def _():
                out_copy(it, b).start()

            @pl.when(bi == _NVB - 1)
            def _():
                out_copy_last(it, b).start()

            @pl.when(blk(it + _A_NBUF) < _NVB)
            def _():
                in_copy(it + _A_NBUF, b).start()

    def group(g, carry):
        for b in range(_A_NBUF):
            step(g * _A_NBUF + b, b)
        return carry

    lax.fori_loop(0, -(-_A_ITERS // _A_NBUF), group, 0)

    last_groups = -(-_A_ITERS // _A_NBUF)
    for b in range(_A_NBUF):
        it = (last_groups - 1) * _A_NBUF + b
        for back in range(2):
            itb = it - back * _A_NBUF
            bi_tb = blk(itb)
            pending = (bi_tb < _NVB) & (bi_tb + _NW * _A_NBUF >= _NVB)

            @pl.when(pending & (bi_tb < _NVB - 1))
            def _(itb=itb, b=b):
                out_copy(itb, b).wait()

            @pl.when(pending & (bi_tb == _NVB - 1))
            def _(itb=itb, b=b):
                out_copy_last(itb, b).wait()


_prep_call = functools.partial(
    pl.kernel,
    out_type=jax.ShapeDtypeStruct((_VOCAB, 2 * _D), jnp.float32),
    mesh=plsc.VectorSubcoreMesh(core_axis_name="c", subcore_axis_name="s"),
    compiler_params=pltpu.CompilerParams(needs_layout_passes=False),
    scratch_types=(
        [pltpu.VMEM((_D, _VBLK + 1), jnp.float32) for _ in range(_A_NBUF)]
        + [pltpu.VMEM((_VBLK, 2 * _D), jnp.float32) for _ in range(_A_NBUF)]
        + [pltpu.SemaphoreType.DMA for _ in range(2 * _A_NBUF)]
    ),
)(_prep_body)


def _lookup_body(tp_hbm, xt_hbm, out_hbm, idx_v, *rest):
    gbufs = rest[0:_B_NBUF]
    obufs = rest[_B_NBUF:_B_NBUF + _B_OBUF]
    gsems = rest[_B_NBUF + _B_OBUF:2 * _B_NBUF + _B_OBUF]
    osems = rest[2 * _B_NBUF + _B_OBUF:2 * _B_NBUF + 2 * _B_OBUF]

    wid = lax.axis_index("s") * _NC + lax.axis_index("c")
    col0 = wid * _BBLK
    iota = lax.iota(jnp.int32, _LANES)

    pltpu.sync_copy(xt_hbm.at[pl.ds(0, _SLAB), pl.ds(col0, _BBLK)], idx_v)

    def gather_copy(c, b):
        return pltpu.make_async_copy(
            tp_hbm.at[idx_v.at[c % _SLAB]], gbufs[b], gsems[b])

    def out_start(c, ob):
        for dt in range(_D // 8):
            pltpu.make_async_copy(
                obufs[ob].at[pl.ds(8 * dt, 8), pl.ds(0, _BBLK)],
                out_hbm.at[c, pl.ds(8 * dt, 8), pl.ds(col0, _BBLK)],
                osems[ob]).start()

    def out_wait(c, ob):
        for dt in range(_D // 8):
            pltpu.make_async_copy(
                obufs[ob].at[pl.ds(8 * dt, 8), pl.ds(0, _BBLK)],
                out_hbm.at[c, pl.ds(8 * dt, 8), pl.ds(col0, _BBLK)],
                osems[ob]).wait()

    for b in range(_B_NBUF):
        gather_copy(b, b).start()

    def group(g, carry):
        @pl.when(((g + 1) % (_SLAB // _B_NBUF) == 0)
                 & (g + 1 < _B_NGROUP))
        def _():
            sidx = (g + 1) // (_SLAB // _B_NBUF)
            pltpu.sync_copy(
                xt_hbm.at[pl.ds(sidx * _SLAB, _SLAB), pl.ds(col0, _BBLK)],
                idx_v)

        for b in range(_B_NBUF):
            c = g * _B_NBUF + b
            ob = b % _B_OBUF
            gather_copy(c, b).wait()

            @pl.when(c >= _B_OBUF)
            def _():
                out_wait(c - _B_OBUF, ob)

            gbuf = gbufs[b]
            obuf = obufs[ob]

            @plsc.parallel_loop(0, _BBLK, unroll=8)
            def _row(j, gbuf=gbuf, obuf=obuf):
                ji = iota * 0 + j
                for kk in range(_D // _LANES):
                    vals = gbuf[j, pl.ds(kk * _LANES, _LANES)]
                    plsc.store_scatter(obuf, [iota + kk * _LANES, ji], vals)

            out_start(c, ob)

            @pl.when(g + 1 < _B_NGROUP)
            def _():
                gather_copy(c + _B_NBUF, b).start()
        return carry

    lax.fori_loop(0, _B_NGROUP, group, 0)

    for b in range(_B_OBUF):
        c = _L - _B_OBUF + b
        out_wait(c, c % _B_OBUF)


_lookup_call = functools.partial(
    pl.kernel,
    out_type=jax.ShapeDtypeStruct((_L, _D, _B), jnp.float32),
    mesh=plsc.VectorSubcoreMesh(core_axis_name="c", subcore_axis_name="s"),
    compiler_params=pltpu.CompilerParams(needs_layout_passes=False),
    scratch_types=(
        [pltpu.VMEM((_SLAB, _BBLK), jnp.int32)]
        + [pltpu.VMEM((_BBLK, 2 * _D), jnp.float32) for _ in range(_B_NBUF)]
        + [pltpu.VMEM((_D, _BBLK + 1), jnp.float32) for _ in range(_B_OBUF)]
        + [pltpu.SemaphoreType.DMA for _ in range(_B_NBUF + _B_OBUF)]
    ),
)(_lookup_body)


@jax.jit
def kernel(embedding_table, x):
    tp = _prep_call(embedding_table.T)
    out = _lookup_call(tp, x.T)
    return out.transpose(2, 0, 1)
